# baseline (speedup 1.0000x reference)
import functools

import jax
import jax.numpy as jnp
from jax import lax
from jax.experimental import pallas as pl
from jax.experimental.pallas import tpu as pltpu
from jax.experimental.pallas import tpu_sc as plsc

_L = 16


def kernel(indices, weight):
    (B,) = indices.shape
    V, D = weight.shape
    info = plsc.get_sparse_core_info()
    num_workers = info.num_cores * info.num_subcores
    per = B // num_workers
    split = (per * 5 // 8) // _L * _L

    idx32 = indices.astype(jnp.int32)
    mesh = plsc.VectorSubcoreMesh(core_axis_name="c", subcore_axis_name="s")

    @functools.partial(
        pl.kernel,
        mesh=mesh,
        out_type=jax.ShapeDtypeStruct((B, D), jnp.float32),
        scratch_types=[
            pltpu.VMEM((per,), jnp.int32),
            pltpu.VMEM((per, 64), jnp.float32),
            pltpu.SemaphoreType.DMA,
            pltpu.SemaphoreType.DMA,
        ],
    )
    def gather_kernel(idx_hbm, table_hbm, out_hbm, idx_v, rows_v, sem_v, sem_h):
        wid = lax.axis_index("s") * info.num_cores + lax.axis_index("c")
        base = wid * per
        pltpu.sync_copy(idx_hbm.at[pl.ds(base, per)], idx_v)

        def body_vmem(g, carry):
            vec = idx_v[pl.ds(g * _L, _L)]
            for j in range(_L):
                row = vec[j]
                pltpu.async_copy(
                    table_hbm.at[pl.ds(row, 1)],
                    rows_v.at[pl.ds(g * _L + j, 1)],
                    sem_v,
                )
            return carry

        def body_hbm(g, carry):
            vec = idx_v[pl.ds(g * _L, _L)]
            for j in range(_L):
                row = vec[j]
                pltpu.async_copy(
                    table_hbm.at[pl.ds(row, 1)],
                    out_hbm.at[pl.ds(base + g * _L + j, 1)],
                    sem_h,
                )
            return carry

        lax.fori_loop(0, split // _L, body_vmem, 0)
        lax.fori_loop(split // _L, per // _L, body_hbm, 0)

        pltpu.make_async_copy(
            table_hbm.at[pl.ds(0, split)], rows_v.at[pl.ds(0, split)], sem_v
        ).wait()
        pltpu.sync_copy(
            rows_v.at[pl.ds(0, split)], out_hbm.at[pl.ds(base, split)]
        )
        pltpu.make_async_copy(
            table_hbm.at[pl.ds(0, per - split)],
            out_hbm.at[pl.ds(base, per - split)],
            sem_h,
        ).wait()

    return gather_kernel(idx32, weight)


# speedup vs baseline: 1.2473x; 1.2473x over previous
import functools

import jax
import jax.numpy as jnp
from jax import lax
from jax.experimental import pallas as pl
from jax.experimental.pallas import tpu as pltpu
from jax.experimental.pallas import tpu_sc as plsc

_L = 16
_NSEM = 4


def kernel(indices, weight):
    (B,) = indices.shape
    V, D = weight.shape
    info = plsc.get_sparse_core_info()
    num_workers = info.num_cores * info.num_subcores
    per = B // num_workers

    idx32 = indices.astype(jnp.int32)
    mesh = plsc.VectorSubcoreMesh(core_axis_name="c", subcore_axis_name="s")

    @functools.partial(
        pl.kernel,
        mesh=mesh,
        out_type=jax.ShapeDtypeStruct((B, D), jnp.float32),
        scratch_types=[
            pltpu.VMEM((per,), jnp.int32),
            pltpu.VMEM((per, 64), jnp.float32),
            [pltpu.SemaphoreType.DMA] * _NSEM,
        ],
    )
    def gather_kernel(idx_hbm, table_hbm, out_hbm, idx_v, rows_v, sems):
        wid = lax.axis_index("s") * info.num_cores + lax.axis_index("c")
        base = wid * per
        pltpu.sync_copy(idx_hbm.at[pl.ds(base, per)], idx_v)

        def body(g, carry):
            vec = idx_v[pl.ds(g * _L, _L)]
            for j in range(_L):
                row = vec[j]
                pltpu.async_copy(
                    table_hbm.at[pl.ds(row, 1)],
                    rows_v.at[pl.ds(g * _L + j, 1)],
                    sems[j % _NSEM],
                )
            return carry

        lax.fori_loop(0, per // _L, body, 0)
        for k in range(_NSEM):
            pltpu.make_async_copy(
                table_hbm.at[pl.ds(0, per // _NSEM)],
                rows_v.at[pl.ds(0, per // _NSEM)],
                sems[k],
            ).wait()
        pltpu.sync_copy(rows_v, out_hbm.at[pl.ds(base, per)])

    return gather_kernel(idx32, weight)


# speedup vs baseline: 2.0796x; 1.6673x over previous
import functools

import jax
import jax.numpy as jnp
from jax import lax
from jax.experimental import pallas as pl
from jax.experimental.pallas import tpu as pltpu
from jax.experimental.pallas import tpu_sc as plsc

_L = 16
_NB = 8


def kernel(indices, weight):
    (B,) = indices.shape
    V, D = weight.shape
    info = plsc.get_sparse_core_info()
    num_workers = info.num_cores * info.num_subcores
    per = B // num_workers

    idx32 = indices.astype(jnp.int32)
    wT = weight.T
    mesh = plsc.VectorSubcoreMesh(core_axis_name="c", subcore_axis_name="s")

    @functools.partial(
        pl.kernel,
        mesh=mesh,
        out_type=jax.ShapeDtypeStruct((D, B), jnp.float32),
        scratch_types=[
            pltpu.VMEM((per,), jnp.int32),
            pltpu.VMEM((_NB, D, 128), jnp.float32),
            pltpu.VMEM((D, per), jnp.float32),
            pltpu.SemaphoreType.DMA,
        ],
        compiler_params=pltpu.CompilerParams(needs_layout_passes=False),
    )
    def gather_kernel(idx_hbm, table_hbm, out_hbm, idx_v, ring, cols_v, sem):
        wid = lax.axis_index("s") * info.num_cores + lax.axis_index("c")
        base = wid * per
        pltpu.sync_copy(idx_hbm.at[pl.ds(base, per)], idx_v)

        lane16 = lax.iota(jnp.int32, _L)

        def fetch(block_start, slot):
            pltpu.async_copy(
                table_hbm.at[:, pl.ds(pl.multiple_of(block_start, 128), 128)],
                ring.at[slot],
                sem,
            )

        def wait_fetch(slot):
            pltpu.make_async_copy(
                table_hbm.at[:, pl.ds(0, 128)], ring.at[slot], sem
            ).wait()

        def body(g, carry):
            vec = idx_v[pl.ds(g * _L, _L)]
            starts = (vec >> 7) << 7
            lanes = vec & 127
            for j in range(_NB):
                fetch(starts[j], j)
            for j in range(_L):
                wait_fetch(j % _NB)
                lane = lanes[j]
                pos = g * _L + j
                for c in range(D // _L):
                    sub = lane16 + c * _L
                    val = plsc.load_gather(
                        ring.at[j % _NB], [sub, lax.broadcast(lane, (_L,))]
                    )
                    plsc.store_scatter(
                        cols_v, [sub, lax.broadcast(pos, (_L,))], val
                    )
                if j + _NB < _L:
                    fetch(starts[j + _NB], (j + _NB) % _NB)
            return carry

        lax.fori_loop(0, per // _L, body, 0)
        pltpu.sync_copy(cols_v, out_hbm.at[:, pl.ds(base, per)])

    outT = gather_kernel(idx32, wT)
    return outT.T
